# baseline (speedup 1.0000x reference)
import functools

import jax
import jax.numpy as jnp
from jax import lax
from jax.experimental import pallas as pl
from jax.experimental.pallas import tpu as pltpu
from jax.experimental.pallas import tpu_sc as plsc

VOCAB = 1000000
EMBED_DIM = 64
BATCH = 16384

_NC = 2
_NS = 16
_NW = _NC * _NS
_BPW = BATCH // _NW
_LANES = 16


def _sc_skipgram(target, context, emb_weight):
    mesh = plsc.VectorSubcoreMesh(core_axis_name="c", subcore_axis_name="s")

    @functools.partial(
        pl.kernel,
        mesh=mesh,
        out_type=jax.ShapeDtypeStruct((BATCH,), jnp.float32),
        compiler_params=pltpu.CompilerParams(
            needs_layout_passes=False, use_tc_tiling_on_sc=False),
        scratch_types=[
            pltpu.VMEM((_BPW,), jnp.int32),
            pltpu.VMEM((_BPW,), jnp.int32),
            pltpu.VMEM((_BPW, EMBED_DIM), jnp.float32),
            pltpu.VMEM((_BPW, EMBED_DIM), jnp.float32),
            pltpu.VMEM((_BPW,), jnp.float32),
            pltpu.SemaphoreType.DMA,
            pltpu.SemaphoreType.DMA,
        ],
    )
    def k(tgt_hbm, ctx_hbm, table_hbm, out_hbm,
          idx_t, idx_c, rows_t, rows_c, scores, sem_t, sem_c):
        wid = lax.axis_index("s") * _NC + lax.axis_index("c")
        base = wid * _BPW

        pltpu.sync_copy(tgt_hbm.at[pl.ds(base, _BPW)], idx_t)
        pltpu.sync_copy(ctx_hbm.at[pl.ds(base, _BPW)], idx_c)

        cp_t = pltpu.async_copy(table_hbm.at[idx_t], rows_t, sem_t)
        cp_c = pltpu.async_copy(table_hbm.at[idx_c], rows_c, sem_c)
        cp_t.wait()
        cp_c.wait()

        lane = lax.iota(jnp.int32, _LANES)

        def chunk_body(i, _):
            vec = jnp.zeros((_LANES,), jnp.float32)
            for j in range(_LANES):
                r = i * _LANES + j
                acc = jnp.zeros((_LANES,), jnp.float32)
                for k_ in range(EMBED_DIM // _LANES):
                    t = rows_t[r, pl.ds(k_ * _LANES, _LANES)]
                    c = rows_c[r, pl.ds(k_ * _LANES, _LANES)]
                    acc = acc + t * c
                vec = jnp.where(lane == j, jnp.sum(acc), vec)
            scores[pl.ds(i * _LANES, _LANES)] = vec
            return 0

        lax.fori_loop(0, _BPW // _LANES, chunk_body, 0)

        pltpu.sync_copy(scores, out_hbm.at[pl.ds(base, _BPW)])

    return k(target, context, emb_weight)


def kernel(target, context, emb_weight):
    return _sc_skipgram(target.astype(jnp.int32), context.astype(jnp.int32),
                        emb_weight)
